# baseline (speedup 1.0000x reference)
import functools

import numpy as np
import jax
import jax.numpy as jnp
from jax import lax
from jax.experimental import pallas as pl
from jax.experimental.pallas import tpu as pltpu
from jax.experimental.pallas import tpu_sc as plsc

N = 10000
E = 160000
G = 256
EMB = 128
D_EDGE = 16
NB = 3
CUTOFF = 10.0

NC = 2
NS = 16
NW = NC * NS
CH = 64
E_PAD = 163840
EW = E_PAD // NW
NCH = EW // CH
PCH = 128
N_ACC = 10240
STRIPE = N_ACC // NS
DUMMY = N

_OFF_NP = np.zeros((1, 64), np.float32)
_OFF_NP[0, :50] = np.linspace(0.0, CUTOFF, 50, dtype=np.float32)
_COEFF = float(-0.5 / (_OFF_NP[0, 1] - _OFF_NP[0, 0]) ** 2)
_LOG2 = float(np.log(2.0))

@functools.lru_cache(maxsize=None)
def _get_mesh():
    return plsc.VectorSubcoreMesh(core_axis_name="c", subcore_axis_name="s",
                                  num_cores=NC, num_subcores=NS)


def _dot(a, b):
    return jnp.dot(a, b, preferred_element_type=jnp.float32)


def _ssp(v):
    return jnp.maximum(v, 0.0) + jnp.log1p(jnp.exp(-jnp.abs(v))) - _LOG2


def _bn(h, g, b):
    m = jnp.mean(h, axis=0, keepdims=True)
    v = jnp.mean((h - m) * (h - m), axis=0, keepdims=True)
    return (h - m) / jnp.sqrt(v + 1e-5) * g + b


@functools.lru_cache(maxsize=None)
def _get_sc_posdiff():
    @functools.partial(
        pl.kernel,
        out_type=jax.ShapeDtypeStruct((E_PAD, 16), jnp.float32),
        mesh=_get_mesh(),
        scratch_types=[
            pltpu.VMEM((PCH,), jnp.int32),
            pltpu.VMEM((PCH,), jnp.int32),
            pltpu.VMEM((PCH,), jnp.int32),
            pltpu.VMEM((PCH,), jnp.int32),
            pltpu.VMEM((PCH, 16), jnp.float32),
            pltpu.VMEM((PCH, 16), jnp.float32),
            pltpu.VMEM((PCH, 16), jnp.float32),
            pltpu.VMEM((PCH, 16), jnp.float32),
            pltpu.SemaphoreType.DMA,
            pltpu.SemaphoreType.DMA,
            pltpu.SemaphoreType.DMA,
            pltpu.SemaphoreType.DMA,
        ],
        compiler_params=pltpu.CompilerParams(use_tc_tiling_on_sc=False),
    )
    def _sc_posdiff(pos_hbm, ridx_hbm, cidx_hbm, out_hbm,
                    ridx_a, cidx_a, ridx_b, cidx_b,
                    pr_a, pc_a, pr_b, pc_b,
                    isem_a, isem_b, gsem_a, gsem_b):
        c = lax.axis_index("c")
        s = lax.axis_index("s")
        w = s * NC + c
        npch = EW // PCH

        def idx_copies(k, rbuf, cbuf, sem):
            base = w * EW + k * PCH
            return (pltpu.make_async_copy(ridx_hbm.at[pl.ds(base, PCH)], rbuf, sem),
                    pltpu.make_async_copy(cidx_hbm.at[pl.ds(base, PCH)], cbuf, sem))

        def gath(rbuf, cbuf, prb, pcb, sem):
            return (pltpu.make_async_copy(pos_hbm.at[rbuf], prb, sem),
                    pltpu.make_async_copy(pos_hbm.at[cbuf], pcb, sem))

        def compute(prb, pcb):
            def erow(i, _):
                prb[i] = prb[i] - pcb[i]
                return 0

            lax.fori_loop(0, PCH, erow, 0)

        ia0, ia1 = idx_copies(0, ridx_a, cidx_a, isem_a)
        ia0.start(); ia1.start()
        ib0, ib1 = idx_copies(1, ridx_b, cidx_b, isem_b)
        ib0.start(); ib1.start()
        ia0.wait(); ia1.wait()
        ga0, ga1 = gath(ridx_a, cidx_a, pr_a, pc_a, gsem_a)
        ga0.start(); ga1.start()

        def pair(j, _):
            k0 = j * 2
            jb0, jb1 = idx_copies(k0 + 1, ridx_b, cidx_b, isem_b)
            jb0.wait(); jb1.wait()
            gb0, gb1 = gath(ridx_b, cidx_b, pr_b, pc_b, gsem_b)
            gb0.start(); gb1.start()
            wa0, wa1 = gath(ridx_a, cidx_a, pr_a, pc_a, gsem_a)
            wa0.wait(); wa1.wait()
            compute(pr_a, pc_a)
            pltpu.sync_copy(pr_a, out_hbm.at[pl.ds(w * EW + k0 * PCH, PCH), :])

            @pl.when(j < npch // 2 - 1)
            def _():
                ja0, ja1 = idx_copies(k0 + 2, ridx_a, cidx_a, isem_a)
                ja0.start(); ja1.start()
                ja0.wait(); ja1.wait()
                na0, na1 = gath(ridx_a, cidx_a, pr_a, pc_a, gsem_a)
                na0.start(); na1.start()

            wb0, wb1 = gath(ridx_b, cidx_b, pr_b, pc_b, gsem_b)
            wb0.wait(); wb1.wait()
            compute(pr_b, pc_b)
            pltpu.sync_copy(pr_b, out_hbm.at[pl.ds(w * EW + (k0 + 1) * PCH, PCH), :])

            @pl.when(j < npch // 2 - 1)
            def _():
                jn0, jn1 = idx_copies(k0 + 3, ridx_b, cidx_b, isem_b)
                jn0.start(); jn1.start()
            return 0

        lax.fori_loop(0, npch // 2, pair, 0)

    return _sc_posdiff


@functools.lru_cache(maxsize=None)
def _make_sc_conv(mode):
    @functools.partial(
        pl.kernel,
        out_type=jax.ShapeDtypeStruct((NC, N_ACC, EMB), jnp.float32),
        mesh=_get_mesh(),
        scratch_types=[
            pltpu.VMEM_SHARED((N_ACC, EMB), jnp.float32),
            pltpu.VMEM((CH,), jnp.int32),
            pltpu.VMEM((CH,), jnp.int32),
            pltpu.VMEM((CH,), jnp.int32),
            pltpu.VMEM((CH,), jnp.int32),
            pltpu.VMEM((CH, EMB), jnp.float32),
            pltpu.VMEM((CH, EMB), jnp.float32),
            pltpu.VMEM((CH, EMB), jnp.float32),
            pltpu.VMEM((CH, EMB), jnp.float32),
            pltpu.SemaphoreType.DMA,
            pltpu.SemaphoreType.DMA,
            pltpu.SemaphoreType.DMA,
            pltpu.SemaphoreType.DMA,
            pltpu.SemaphoreType.DMA,
            pltpu.SemaphoreType.DMA,
            pltpu.SemaphoreType.DMA,
            pltpu.SemaphoreType.DMA,
        ],
        name=f"sc_conv_{mode}",
    )
    def sc_conv(table_hbm, feat_hbm, gidx_hbm, sidx_hbm, out_hbm,
                acc, gidx_a, gidx_b, sidx_a, sidx_b,
                rows_a, rows_b, feat_a, feat_b,
                gsem_a, gsem_b, fsem_a, fsem_b, isem_a, isem_b,
                ssem_a, ssem_b):
        c = lax.axis_index("c")
        s = lax.axis_index("s")
        w = s * NC + c
        zero = jnp.zeros((16,), jnp.float32)

        def idx_copies(k, gbuf, sbuf, sem):
            base = w * EW + k * CH
            return (pltpu.make_async_copy(gidx_hbm.at[pl.ds(base, CH)], gbuf, sem),
                    pltpu.make_async_copy(sidx_hbm.at[pl.ds(base, CH)], sbuf, sem))

        def feat_copy(k, buf, sem):
            return pltpu.make_async_copy(
                feat_hbm.at[pl.ds(w * EW + k * CH, CH), :], buf, sem)

        def gath_copy(gbuf, buf, sem):
            return pltpu.make_async_copy(table_hbm.at[gbuf], buf, sem)

        def combine(rbuf, fbuf):
            def erow(i2, _):
                for u in range(2):
                    i = i2 * 2 + u
                    for j in range(EMB // 16):
                        a = rbuf[i, pl.ds(j * 16, 16)]
                        b = fbuf[i, pl.ds(j * 16, 16)]
                        if mode == "relu_add":
                            r = jnp.maximum(a + b, 0.0)
                        else:
                            r = a * b
                        rbuf[i, pl.ds(j * 16, 16)] = r
                return 0

            lax.fori_loop(0, CH // 2, erow, 0)

        def zrow(i, _):
            for j in range(EMB // 16):
                rows_a[i, pl.ds(j * 16, 16)] = zero
            return 0

        lax.fori_loop(0, CH, zrow, 0)
        for j in range(STRIPE // CH):
            pltpu.sync_copy(rows_a, acc.at[pl.ds(s * STRIPE + j * CH, CH), :])
        plsc.subcore_barrier()

        ia0, ia1 = idx_copies(0, gidx_a, sidx_a, isem_a)
        ia0.start(); ia1.start()
        ib0, ib1 = idx_copies(1, gidx_b, sidx_b, isem_b)
        ib0.start(); ib1.start()
        ia0.wait(); ia1.wait()
        feat_copy(0, feat_a, fsem_a).start()
        gath_copy(gidx_a, rows_a, gsem_a).start()

        def pair(j, _):
            k0 = j * 2
            jb0, jb1 = idx_copies(k0 + 1, gidx_b, sidx_b, isem_b)
            jb0.wait(); jb1.wait()
            feat_copy(k0 + 1, feat_b, fsem_b).start()
            gath_copy(gidx_b, rows_b, gsem_b).start()
            feat_copy(k0, feat_a, fsem_a).wait()
            gath_copy(gidx_a, rows_a, gsem_a).wait()
            combine(rows_a, feat_a)
            pltpu.sync_copy(rows_a, acc.at[sidx_a], add=True)

            @pl.when(j < NCH // 2 - 1)
            def _():
                ja0, ja1 = idx_copies(k0 + 2, gidx_a, sidx_a, isem_a)
                ja0.start(); ja1.start()
                ja0.wait(); ja1.wait()
                feat_copy(k0 + 2, feat_a, fsem_a).start()
                gath_copy(gidx_a, rows_a, gsem_a).start()

            feat_copy(k0 + 1, feat_b, fsem_b).wait()
            gath_copy(gidx_b, rows_b, gsem_b).wait()
            combine(rows_b, feat_b)
            pltpu.sync_copy(rows_b, acc.at[sidx_b], add=True)

            @pl.when(j < NCH // 2 - 1)
            def _():
                jn0, jn1 = idx_copies(k0 + 3, gidx_b, sidx_b, isem_b)
                jn0.start(); jn1.start()
            return 0

        lax.fori_loop(0, NCH // 2, pair, 0)
        plsc.subcore_barrier()
        for j in range(STRIPE // CH):
            r0 = s * STRIPE + j * CH
            pltpu.sync_copy(acc.at[pl.ds(r0, CH), :], rows_a)
            pltpu.sync_copy(rows_a, out_hbm.at[c, pl.ds(r0, CH), :])

    return sc_conv


def _sc_posdiff(*args):
    return _get_sc_posdiff()(*args)


def _sc_conv_add(*args):
    return _make_sc_conv("relu_add")(*args)


def _sc_conv_mul(*args):
    return _make_sc_conv("mul")(*args)


_BR = 2048


def _tc_edge_body(offs_ref, diff_ref, attr_ref, geW_ref, geb_ref, fW1_ref,
                  fb1_ref, fW2_ref, fb2_ref, eemb_ref, wf_ref):
    diff = diff_ref[...]
    d2 = jnp.sum(diff * diff, axis=1, keepdims=True)
    d = jnp.sqrt(d2 + 1e-12)
    offs = offs_ref[...]
    ea3 = jnp.exp(_COEFF * (d - offs) ** 2)
    ccut = 0.5 * (jnp.cos(d * (np.pi / CUTOFF)) + 1.0)
    ccut = ccut * (d < CUTOFF).astype(jnp.float32)
    attr = attr_ref[...]
    for i in range(NB):
        e = _dot(attr, geW_ref[i])
        eemb_ref[i] = e + geb_ref[i][None, :]
        f = _ssp(_dot(ea3, fW1_ref[i])
                 + fb1_ref[i][None, :])
        wv = _dot(f, fW2_ref[i])
        wf_ref[i] = (wv + fb2_ref[i][None, :]) * ccut


def _tc_edge(offs, diff, attr, geW, geb, fW1, fb1, fW2, fb2):
    grid = (E_PAD // _BR,)
    full = lambda shape: pl.BlockSpec(shape, lambda e: (0,) * len(shape))
    return pl.pallas_call(
        _tc_edge_body,
        grid=grid,
        in_specs=[
            full((1, 64)),
            pl.BlockSpec((_BR, 16), lambda e: (e, 0)),
            pl.BlockSpec((_BR, D_EDGE), lambda e: (e, 0)),
            full((NB, D_EDGE, EMB)),
            full((NB, EMB)),
            full((NB, 64, EMB)),
            full((NB, EMB)),
            full((NB, EMB, EMB)),
            full((NB, EMB)),
        ],
        out_specs=[
            pl.BlockSpec((NB, _BR, EMB), lambda e: (0, e, 0)),
            pl.BlockSpec((NB, _BR, EMB), lambda e: (0, e, 0)),
        ],
        out_shape=[
            jax.ShapeDtypeStruct((NB, E_PAD, EMB), jnp.float32),
            jax.ShapeDtypeStruct((NB, E_PAD, EMB), jnp.float32),
        ],
    )(offs, diff, attr, geW, geb, fW1, fb1, fW2, fb2)


def _tc_init_body(x_ref, batch_ref, emb2_ref, emb3_ref, lin1_ref,
                  x2_ref, x3_ref, h3_ref, vi_ref):
    lanes = lax.broadcasted_iota(jnp.int32, (N, EMB), 1)
    oh = (x_ref[...] == lanes).astype(jnp.float32)
    x2 = jnp.dot(oh, emb2_ref[...], preferred_element_type=jnp.float32,
                 precision=lax.Precision.HIGHEST)
    x3 = jnp.dot(oh, emb3_ref[...], preferred_element_type=jnp.float32,
                 precision=lax.Precision.HIGHEST)
    x2_ref[...] = x2
    x3_ref[...] = x3
    h3_ref[...] = _dot(x3, lin1_ref[...])
    gi = lax.broadcasted_iota(jnp.int32, (N, G), 1)
    m = (batch_ref[...] <= gi).astype(jnp.float32)
    cnt = jnp.sum(m, axis=0, keepdims=True)
    vi_ref[...] = jnp.clip(cnt.astype(jnp.int32) - 1, 0, N - 1)


def _tc_init(x2col, batch2col, emb2, emb3, lin1):
    return pl.pallas_call(
        _tc_init_body,
        out_shape=[
            jax.ShapeDtypeStruct((N, EMB), jnp.float32),
            jax.ShapeDtypeStruct((N, EMB), jnp.float32),
            jax.ShapeDtypeStruct((N, EMB), jnp.float32),
            jax.ShapeDtypeStruct((1, G), jnp.int32),
        ],
    )(x2col, batch2col, emb2, emb3, lin1)


def _tc_node2d_body(x2_ref, prev_ref, p_ref, W1_ref, b1_ref, W2_ref, b2_ref,
                    g_ref, b_ref, out_ref):
    p = p_ref[...]
    aggr = p[0, :N, :] + p[1, :N, :]
    h = x2_ref[...] + aggr
    h = jnp.maximum(_dot(h, W1_ref[...])
                    + b1_ref[...], 0.0)
    h = _dot(h, W2_ref[...]) + b2_ref[...]
    r = jnp.maximum(h, 0.0)
    out_ref[...] = _bn(r, g_ref[...], b_ref[...]) + prev_ref[...]


def _tc_node2d(x2, prev, p, W1, b1, W2, b2, g, b):
    return pl.pallas_call(
        _tc_node2d_body,
        out_shape=jax.ShapeDtypeStruct((N, EMB), jnp.float32),
    )(x2, prev, p, W1, b1, W2, b2, g, b)


def _tc_node3d_body(prev_ref, p_ref, W2_ref, b2_ref, Wo_ref, bo_ref,
                    g_ref, b_ref, out_ref):
    p = p_ref[...]
    a3 = p[0, :N, :] + p[1, :N, :]
    t = _ssp(_dot(a3, W2_ref[...])
             + b2_ref[...])
    t = _dot(t, Wo_ref[...]) + bo_ref[...]
    r = jnp.maximum(t, 0.0)
    out_ref[...] = _bn(r, g_ref[...], b_ref[...]) + prev_ref[...]


def _tc_node3d(prev, p, W2, b2, Wo, bo, g, b):
    return pl.pallas_call(
        _tc_node3d_body,
        out_shape=jax.ShapeDtypeStruct((N, EMB), jnp.float32),
    )(prev, p, W2, b2, Wo, bo, g, b)


def _tc_virt_body(x2_ref, x3_ref, vi_ref, W1_ref, b1_ref, g_ref, bb_ref,
                  W2_ref, b2_ref, lin1n_ref,
                  x2o_ref, x3o_ref, h3n_ref, out_ref, v2_s, v3_s):
    def gather_row(g, _):
        v = vi_ref[0, g]
        v2_s[pl.ds(g, 1), :] = x2_ref[pl.ds(v, 1), :]
        v3_s[pl.ds(g, 1), :] = x3_ref[pl.ds(v, 1), :]
        return 0

    lax.fori_loop(0, G, gather_row, 0)
    v2 = v2_s[...]
    v3 = v3_s[...]
    inter = jnp.concatenate([v2, v3], axis=1)
    t = _dot(inter, W1_ref[...]) + b1_ref[...]
    t = jnp.maximum(_bn(t, g_ref[...], bb_ref[...]), 0.0)
    t = _dot(t, W2_ref[...]) + b2_ref[...]
    x2o_ref[...] = x2_ref[...]
    x3o_ref[...] = x3_ref[...]
    v2_s[...] = t[:, :EMB]
    v3_s[...] = t[:, EMB:]

    def scatter_row(g, _):
        v = vi_ref[0, g]
        x2o_ref[pl.ds(v, 1), :] = v2_s[pl.ds(g, 1), :]
        x3o_ref[pl.ds(v, 1), :] = v3_s[pl.ds(g, 1), :]
        return 0

    lax.fori_loop(0, G, scatter_row, 0)
    h3n_ref[...] = _dot(x3o_ref[...], lin1n_ref[...])
    o = _dot(t, W1_ref[...]) + b1_ref[...]
    o = jnp.maximum(_bn(o, g_ref[...], bb_ref[...]), 0.0)
    out_ref[...] = _dot(o, W2_ref[...]) + b2_ref[...]


def _tc_virt(x2, x3, vi, W1, b1, g, bb, W2, b2, lin1n):
    smem = pl.BlockSpec(memory_space=pltpu.SMEM)
    return pl.pallas_call(
        _tc_virt_body,
        in_specs=[pl.BlockSpec((N, EMB), lambda: (0, 0)),
                  pl.BlockSpec((N, EMB), lambda: (0, 0)),
                  smem,
                  pl.BlockSpec((2 * EMB, 2 * EMB), lambda: (0, 0)),
                  pl.BlockSpec((1, 2 * EMB), lambda: (0, 0)),
                  pl.BlockSpec((1, 2 * EMB), lambda: (0, 0)),
                  pl.BlockSpec((1, 2 * EMB), lambda: (0, 0)),
                  pl.BlockSpec((2 * EMB, 2 * EMB), lambda: (0, 0)),
                  pl.BlockSpec((1, 2 * EMB), lambda: (0, 0)),
                  pl.BlockSpec((EMB, EMB), lambda: (0, 0))],
        scratch_shapes=[pltpu.VMEM((G, EMB), jnp.float32),
                        pltpu.VMEM((G, EMB), jnp.float32)],
        out_shape=[
            jax.ShapeDtypeStruct((N, EMB), jnp.float32),
            jax.ShapeDtypeStruct((N, EMB), jnp.float32),
            jax.ShapeDtypeStruct((N, EMB), jnp.float32),
            jax.ShapeDtypeStruct((G, 2 * EMB), jnp.float32),
        ],
    )(x2, x3, vi, W1, b1, g, bb, W2, b2, lin1n)


def kernel(x, edge_index, edge_attr, positions, batch, atom_emb_2d, atom_emb_3d,
           gin_edge_W, gin_edge_b, gin_W1, gin_b1, gin_W2, gin_b2, bn2d_g,
           bn2d_b, sch_filt_W1, sch_filt_b1, sch_filt_W2, sch_filt_b2,
           sch_lin1_W, sch_lin2_W, sch_lin2_b, sch_out_W, sch_out_b, bn3d_g,
           bn3d_b, int_W1, int_b1, int_bn_g, int_bn_b, int_W2, int_b2):
    f32 = jnp.float32
    row = edge_index[0]
    col = edge_index[1]
    padz = jnp.zeros((E_PAD - E,), jnp.int32)
    padd = jnp.full((E_PAD - E,), DUMMY, jnp.int32)
    row_g = jnp.concatenate([row, padz])
    col_g = jnp.concatenate([col, padz])
    row_s = jnp.concatenate([row, padd])
    col_s = jnp.concatenate([col, padd])
    pos_p = jnp.pad(positions.astype(f32), ((0, 0), (0, 13)))
    attr_p = jnp.pad(edge_attr.astype(f32), ((0, E_PAD - E), (0, 0)))
    emb2_p = jnp.pad(atom_emb_2d.astype(f32), ((0, EMB - 119), (0, 0)))
    emb3_p = jnp.pad(atom_emb_3d.astype(f32), ((0, EMB - 119), (0, 0)))
    fW1_p = jnp.pad(sch_filt_W1.astype(f32), ((0, 0), (0, 14), (0, 0)))

    offs = jnp.pad(jnp.linspace(0.0, CUTOFF, 50).astype(f32), (0, 14))
    offs = offs.reshape(1, 64)
    diff = _sc_posdiff(pos_p, row_g, col_g)
    eemb, wf = _tc_edge(offs, diff, attr_p, gin_edge_W.astype(f32),
                        gin_edge_b.astype(f32), fW1_p, sch_filt_b1.astype(f32),
                        sch_filt_W2.astype(f32), sch_filt_b2.astype(f32))
    x2d, x3d, h3, vi = _tc_init(x.reshape(N, 1), batch.reshape(N, 1),
                                emb2_p, emb3_p, sch_lin1_W[0].astype(f32))
    prev2, prev3 = x2d, x3d

    out = None
    for i in range(NB):
        p2 = _sc_conv_add(x2d, eemb[i], row_g, col_s)
        x2m = _tc_node2d(x2d, prev2, p2,
                         gin_W1[i].astype(f32), gin_b1[i].reshape(1, 2 * EMB),
                         gin_W2[i].astype(f32), gin_b2[i].reshape(1, EMB),
                         bn2d_g[i].reshape(1, EMB), bn2d_b[i].reshape(1, EMB))
        p3 = _sc_conv_mul(h3, wf[i], col_g, row_s)
        x3m = _tc_node3d(prev3, p3,
                         sch_lin2_W[i].astype(f32), sch_lin2_b[i].reshape(1, EMB),
                         sch_out_W[i].astype(f32), sch_out_b[i].reshape(1, EMB),
                         bn3d_g[i].reshape(1, EMB), bn3d_b[i].reshape(1, EMB))
        x2d, x3d, h3, out = _tc_virt(
            x2m, x3m, vi, int_W1.astype(f32), int_b1.reshape(1, 2 * EMB),
            int_bn_g.reshape(1, 2 * EMB), int_bn_b.reshape(1, 2 * EMB),
            int_W2.astype(f32), int_b2.reshape(1, 2 * EMB),
            sch_lin1_W[(i + 1) % NB].astype(f32))
    return out


# speedup vs baseline: 1.1605x; 1.1605x over previous
import functools

import numpy as np
import jax
import jax.numpy as jnp
from jax import lax
from jax.experimental import pallas as pl
from jax.experimental.pallas import tpu as pltpu
from jax.experimental.pallas import tpu_sc as plsc

N = 10000
E = 160000
G = 256
EMB = 128
D_EDGE = 16
NB = 3
CUTOFF = 10.0

NC = 2
NS = 16
NW = NC * NS
CH = 64
E_PAD = 163840
EW = E_PAD // NW
NCH = EW // CH
PCH = 128
N_ACC = 10240
STRIPE = N_ACC // NS
DUMMY = N

_OFF_NP = np.zeros((1, 64), np.float32)
_OFF_NP[0, :50] = np.linspace(0.0, CUTOFF, 50, dtype=np.float32)
_COEFF = float(-0.5 / (_OFF_NP[0, 1] - _OFF_NP[0, 0]) ** 2)
_LOG2 = float(np.log(2.0))

@functools.lru_cache(maxsize=None)
def _get_mesh():
    return plsc.VectorSubcoreMesh(core_axis_name="c", subcore_axis_name="s",
                                  num_cores=NC, num_subcores=NS)


def _dot(a, b):
    return jnp.dot(a, b, preferred_element_type=jnp.float32)


def _ssp(v):
    return jnp.maximum(v, 0.0) + jnp.log1p(jnp.exp(-jnp.abs(v))) - _LOG2


def _bn(h, g, b):
    m = jnp.mean(h, axis=0, keepdims=True)
    v = jnp.mean((h - m) * (h - m), axis=0, keepdims=True)
    return (h - m) / jnp.sqrt(v + 1e-5) * g + b


@functools.lru_cache(maxsize=None)
def _get_sc_posdiff():
    @functools.partial(
        pl.kernel,
        out_type=jax.ShapeDtypeStruct((E_PAD, 16), jnp.float32),
        mesh=_get_mesh(),
        scratch_types=[
            pltpu.VMEM((PCH,), jnp.int32),
            pltpu.VMEM((PCH,), jnp.int32),
            pltpu.VMEM((PCH,), jnp.int32),
            pltpu.VMEM((PCH,), jnp.int32),
            pltpu.VMEM((PCH, 16), jnp.float32),
            pltpu.VMEM((PCH, 16), jnp.float32),
            pltpu.VMEM((PCH, 16), jnp.float32),
            pltpu.VMEM((PCH, 16), jnp.float32),
            pltpu.SemaphoreType.DMA,
            pltpu.SemaphoreType.DMA,
            pltpu.SemaphoreType.DMA,
            pltpu.SemaphoreType.DMA,
        ],
        compiler_params=pltpu.CompilerParams(use_tc_tiling_on_sc=False),
    )
    def _sc_posdiff(pos_hbm, ridx_hbm, cidx_hbm, out_hbm,
                    ridx_a, cidx_a, ridx_b, cidx_b,
                    pr_a, pc_a, pr_b, pc_b,
                    isem_a, isem_b, gsem_a, gsem_b):
        c = lax.axis_index("c")
        s = lax.axis_index("s")
        w = s * NC + c
        npch = EW // PCH

        def idx_copies(k, rbuf, cbuf, sem):
            base = w * EW + k * PCH
            return (pltpu.make_async_copy(ridx_hbm.at[pl.ds(base, PCH)], rbuf, sem),
                    pltpu.make_async_copy(cidx_hbm.at[pl.ds(base, PCH)], cbuf, sem))

        def gath(rbuf, cbuf, prb, pcb, sem):
            return (pltpu.make_async_copy(pos_hbm.at[rbuf], prb, sem),
                    pltpu.make_async_copy(pos_hbm.at[cbuf], pcb, sem))

        def compute(prb, pcb):
            def erow(i, _):
                prb[i] = prb[i] - pcb[i]
                return 0

            lax.fori_loop(0, PCH, erow, 0)

        ia0, ia1 = idx_copies(0, ridx_a, cidx_a, isem_a)
        ia0.start(); ia1.start()
        ib0, ib1 = idx_copies(1, ridx_b, cidx_b, isem_b)
        ib0.start(); ib1.start()
        ia0.wait(); ia1.wait()
        ga0, ga1 = gath(ridx_a, cidx_a, pr_a, pc_a, gsem_a)
        ga0.start(); ga1.start()

        def pair(j, _):
            k0 = j * 2
            jb0, jb1 = idx_copies(k0 + 1, ridx_b, cidx_b, isem_b)
            jb0.wait(); jb1.wait()
            gb0, gb1 = gath(ridx_b, cidx_b, pr_b, pc_b, gsem_b)
            gb0.start(); gb1.start()
            wa0, wa1 = gath(ridx_a, cidx_a, pr_a, pc_a, gsem_a)
            wa0.wait(); wa1.wait()
            compute(pr_a, pc_a)
            pltpu.sync_copy(pr_a, out_hbm.at[pl.ds(w * EW + k0 * PCH, PCH), :])

            @pl.when(j < npch // 2 - 1)
            def _():
                ja0, ja1 = idx_copies(k0 + 2, ridx_a, cidx_a, isem_a)
                ja0.start(); ja1.start()
                ja0.wait(); ja1.wait()
                na0, na1 = gath(ridx_a, cidx_a, pr_a, pc_a, gsem_a)
                na0.start(); na1.start()

            wb0, wb1 = gath(ridx_b, cidx_b, pr_b, pc_b, gsem_b)
            wb0.wait(); wb1.wait()
            compute(pr_b, pc_b)
            pltpu.sync_copy(pr_b, out_hbm.at[pl.ds(w * EW + (k0 + 1) * PCH, PCH), :])

            @pl.when(j < npch // 2 - 1)
            def _():
                jn0, jn1 = idx_copies(k0 + 3, ridx_b, cidx_b, isem_b)
                jn0.start(); jn1.start()
            return 0

        lax.fori_loop(0, npch // 2, pair, 0)

    return _sc_posdiff


EW2 = E_PAD // NS
NCH2 = EW2 // CH


@functools.lru_cache(maxsize=None)
def _get_sc_dualconv():
    @functools.partial(
        pl.kernel,
        out_type=jax.ShapeDtypeStruct((NC, N_ACC, EMB), jnp.float32),
        mesh=_get_mesh(),
        scratch_types=[
            pltpu.VMEM_SHARED((N_ACC, EMB), jnp.float32),
            pltpu.VMEM((CH,), jnp.int32),
            pltpu.VMEM((CH,), jnp.int32),
            pltpu.VMEM((CH,), jnp.int32),
            pltpu.VMEM((CH,), jnp.int32),
            pltpu.VMEM((CH, EMB), jnp.float32),
            pltpu.VMEM((CH, EMB), jnp.float32),
            pltpu.VMEM((CH, EMB), jnp.float32),
            pltpu.VMEM((CH, EMB), jnp.float32),
            pltpu.SemaphoreType.DMA,
            pltpu.SemaphoreType.DMA,
            pltpu.SemaphoreType.DMA,
            pltpu.SemaphoreType.DMA,
            pltpu.SemaphoreType.DMA,
            pltpu.SemaphoreType.DMA,
        ],
        name="sc_dualconv",
    )
    def sc_dualconv(t2d_hbm, f2d_hbm, t3d_hbm, f3d_hbm,
                    g2d_hbm, s2d_hbm, g3d_hbm, s3d_hbm, out_hbm,
                    acc, gidx_a, gidx_b, sidx_a, sidx_b,
                    rows_a, rows_b, feat_a, feat_b,
                    gsem_a, gsem_b, fsem_a, fsem_b, isem_a, isem_b):
        c = lax.axis_index("c")
        s = lax.axis_index("s")
        zero = jnp.zeros((16,), jnp.float32)

        def zrow(i, _):
            for j in range(EMB // 16):
                rows_a[i, pl.ds(j * 16, 16)] = zero
            return 0

        lax.fori_loop(0, CH, zrow, 0)
        for j in range(STRIPE // CH):
            pltpu.sync_copy(rows_a, acc.at[pl.ds(s * STRIPE + j * CH, CH), :])
        plsc.subcore_barrier()

        def run_conv(table_hbm, feat_hbm, gidx_hbm, sidx_hbm, mode):
            def idx_copies(k, gbuf, sbuf, sem):
                base = s * EW2 + k * CH
                return (pltpu.make_async_copy(gidx_hbm.at[pl.ds(base, CH)],
                                              gbuf, sem),
                        pltpu.make_async_copy(sidx_hbm.at[pl.ds(base, CH)],
                                              sbuf, sem))

            def feat_copy(k, buf, sem):
                return pltpu.make_async_copy(
                    feat_hbm.at[pl.ds(s * EW2 + k * CH, CH), :], buf, sem)

            def gath_copy(gbuf, buf, sem):
                return pltpu.make_async_copy(table_hbm.at[gbuf], buf, sem)

            def combine(rbuf, fbuf):
                def erow(i, _):
                    for j in range(EMB // 16):
                        a = rbuf[i, pl.ds(j * 16, 16)]
                        b = fbuf[i, pl.ds(j * 16, 16)]
                        if mode == "relu_add":
                            r = jnp.maximum(a + b, 0.0)
                        else:
                            r = a * b
                        rbuf[i, pl.ds(j * 16, 16)] = r
                    return 0

                lax.fori_loop(0, CH, erow, 0)

            ia0, ia1 = idx_copies(0, gidx_a, sidx_a, isem_a)
            ia0.start(); ia1.start()
            ib0, ib1 = idx_copies(1, gidx_b, sidx_b, isem_b)
            ib0.start(); ib1.start()
            ia0.wait(); ia1.wait()
            feat_copy(0, feat_a, fsem_a).start()
            gath_copy(gidx_a, rows_a, gsem_a).start()

            def pair(j, _):
                k0 = j * 2
                jb0, jb1 = idx_copies(k0 + 1, gidx_b, sidx_b, isem_b)
                jb0.wait(); jb1.wait()
                feat_copy(k0 + 1, feat_b, fsem_b).start()
                gath_copy(gidx_b, rows_b, gsem_b).start()
                feat_copy(k0, feat_a, fsem_a).wait()
                gath_copy(gidx_a, rows_a, gsem_a).wait()
                combine(rows_a, feat_a)
                pltpu.sync_copy(rows_a, acc.at[sidx_a], add=True)

                @pl.when(j < NCH2 // 2 - 1)
                def _():
                    ja0, ja1 = idx_copies(k0 + 2, gidx_a, sidx_a, isem_a)
                    ja0.start(); ja1.start()
                    ja0.wait(); ja1.wait()
                    feat_copy(k0 + 2, feat_a, fsem_a).start()
                    gath_copy(gidx_a, rows_a, gsem_a).start()

                feat_copy(k0 + 1, feat_b, fsem_b).wait()
                gath_copy(gidx_b, rows_b, gsem_b).wait()
                combine(rows_b, feat_b)
                pltpu.sync_copy(rows_b, acc.at[sidx_b], add=True)

                @pl.when(j < NCH2 // 2 - 1)
                def _():
                    jn0, jn1 = idx_copies(k0 + 3, gidx_b, sidx_b, isem_b)
                    jn0.start(); jn1.start()
                return 0

            lax.fori_loop(0, NCH2 // 2, pair, 0)

        @pl.when(c == 0)
        def _():
            run_conv(t2d_hbm, f2d_hbm, g2d_hbm, s2d_hbm, "relu_add")

        @pl.when(c == 1)
        def _():
            run_conv(t3d_hbm, f3d_hbm, g3d_hbm, s3d_hbm, "mul")

        plsc.subcore_barrier()
        for j in range(STRIPE // CH):
            r0 = s * STRIPE + j * CH
            pltpu.sync_copy(acc.at[pl.ds(r0, CH), :], rows_a)
            pltpu.sync_copy(rows_a, out_hbm.at[c, pl.ds(r0, CH), :])

    return sc_dualconv


def _sc_posdiff(*args):
    return _get_sc_posdiff()(*args)


def _sc_dualconv(*args):
    return _get_sc_dualconv()(*args)


_BR = 2048


def _tc_edge_body(offs_ref, diff_ref, attr_ref, geW_ref, geb_ref, fW1_ref,
                  fb1_ref, fW2_ref, fb2_ref, eemb_ref, wf_ref):
    diff = diff_ref[...]
    d2 = jnp.sum(diff * diff, axis=1, keepdims=True)
    d = jnp.sqrt(d2 + 1e-12)
    offs = offs_ref[...]
    ea3 = jnp.exp(_COEFF * (d - offs) ** 2)
    ccut = 0.5 * (jnp.cos(d * (np.pi / CUTOFF)) + 1.0)
    ccut = ccut * (d < CUTOFF).astype(jnp.float32)
    attr = attr_ref[...]
    for i in range(NB):
        e = _dot(attr, geW_ref[i])
        eemb_ref[i] = e + geb_ref[i][None, :]
        f = _ssp(_dot(ea3, fW1_ref[i])
                 + fb1_ref[i][None, :])
        wv = _dot(f, fW2_ref[i])
        wf_ref[i] = (wv + fb2_ref[i][None, :]) * ccut


def _tc_edge(offs, diff, attr, geW, geb, fW1, fb1, fW2, fb2):
    grid = (E_PAD // _BR,)
    full = lambda shape: pl.BlockSpec(shape, lambda e: (0,) * len(shape))
    return pl.pallas_call(
        _tc_edge_body,
        grid=grid,
        in_specs=[
            full((1, 64)),
            pl.BlockSpec((_BR, 16), lambda e: (e, 0)),
            pl.BlockSpec((_BR, D_EDGE), lambda e: (e, 0)),
            full((NB, D_EDGE, EMB)),
            full((NB, EMB)),
            full((NB, 64, EMB)),
            full((NB, EMB)),
            full((NB, EMB, EMB)),
            full((NB, EMB)),
        ],
        out_specs=[
            pl.BlockSpec((NB, _BR, EMB), lambda e: (0, e, 0)),
            pl.BlockSpec((NB, _BR, EMB), lambda e: (0, e, 0)),
        ],
        out_shape=[
            jax.ShapeDtypeStruct((NB, E_PAD, EMB), jnp.float32),
            jax.ShapeDtypeStruct((NB, E_PAD, EMB), jnp.float32),
        ],
    )(offs, diff, attr, geW, geb, fW1, fb1, fW2, fb2)


def _tc_init_body(x_ref, batch_ref, emb2_ref, emb3_ref, lin1_ref,
                  x2_ref, x3_ref, h3_ref, vi_ref):
    lanes = lax.broadcasted_iota(jnp.int32, (N, EMB), 1)
    oh = (x_ref[...] == lanes).astype(jnp.float32)
    x2 = jnp.dot(oh, emb2_ref[...], preferred_element_type=jnp.float32,
                 precision=lax.Precision.HIGHEST)
    x3 = jnp.dot(oh, emb3_ref[...], preferred_element_type=jnp.float32,
                 precision=lax.Precision.HIGHEST)
    x2_ref[...] = x2
    x3_ref[...] = x3
    h3_ref[...] = _dot(x3, lin1_ref[...])
    gi = lax.broadcasted_iota(jnp.int32, (N, G), 1)
    m = (batch_ref[...] <= gi).astype(jnp.float32)
    cnt = jnp.sum(m, axis=0, keepdims=True)
    vi_ref[...] = jnp.clip(cnt.astype(jnp.int32) - 1, 0, N - 1)


def _tc_init(x2col, batch2col, emb2, emb3, lin1):
    return pl.pallas_call(
        _tc_init_body,
        out_shape=[
            jax.ShapeDtypeStruct((N, EMB), jnp.float32),
            jax.ShapeDtypeStruct((N, EMB), jnp.float32),
            jax.ShapeDtypeStruct((N, EMB), jnp.float32),
            jax.ShapeDtypeStruct((1, G), jnp.int32),
        ],
    )(x2col, batch2col, emb2, emb3, lin1)


def _tc_node2d_body(x2_ref, prev_ref, p_ref, W1_ref, b1_ref, W2_ref, b2_ref,
                    g_ref, b_ref, out_ref):
    aggr = p_ref[...][:N, :]
    h = x2_ref[...] + aggr
    h = jnp.maximum(_dot(h, W1_ref[...])
                    + b1_ref[...], 0.0)
    h = _dot(h, W2_ref[...]) + b2_ref[...]
    r = jnp.maximum(h, 0.0)
    out_ref[...] = _bn(r, g_ref[...], b_ref[...]) + prev_ref[...]


def _tc_node2d(x2, prev, p, W1, b1, W2, b2, g, b):
    return pl.pallas_call(
        _tc_node2d_body,
        out_shape=jax.ShapeDtypeStruct((N, EMB), jnp.float32),
    )(x2, prev, p, W1, b1, W2, b2, g, b)


def _tc_node3d_body(prev_ref, p_ref, W2_ref, b2_ref, Wo_ref, bo_ref,
                    g_ref, b_ref, out_ref):
    a3 = p_ref[...][:N, :]
    t = _ssp(_dot(a3, W2_ref[...])
             + b2_ref[...])
    t = _dot(t, Wo_ref[...]) + bo_ref[...]
    r = jnp.maximum(t, 0.0)
    out_ref[...] = _bn(r, g_ref[...], b_ref[...]) + prev_ref[...]


def _tc_node3d(prev, p, W2, b2, Wo, bo, g, b):
    return pl.pallas_call(
        _tc_node3d_body,
        out_shape=jax.ShapeDtypeStruct((N, EMB), jnp.float32),
    )(prev, p, W2, b2, Wo, bo, g, b)


def _tc_virt_body(x2_ref, x3_ref, vi_ref, W1_ref, b1_ref, g_ref, bb_ref,
                  W2_ref, b2_ref, lin1n_ref,
                  x2o_ref, x3o_ref, h3n_ref, out_ref, v2_s, v3_s):
    def gather_row(g, _):
        v = vi_ref[0, g]
        v2_s[pl.ds(g, 1), :] = x2_ref[pl.ds(v, 1), :]
        v3_s[pl.ds(g, 1), :] = x3_ref[pl.ds(v, 1), :]
        return 0

    lax.fori_loop(0, G, gather_row, 0)
    v2 = v2_s[...]
    v3 = v3_s[...]
    inter = jnp.concatenate([v2, v3], axis=1)
    t = _dot(inter, W1_ref[...]) + b1_ref[...]
    t = jnp.maximum(_bn(t, g_ref[...], bb_ref[...]), 0.0)
    t = _dot(t, W2_ref[...]) + b2_ref[...]
    x2o_ref[...] = x2_ref[...]
    x3o_ref[...] = x3_ref[...]
    v2_s[...] = t[:, :EMB]
    v3_s[...] = t[:, EMB:]

    def scatter_row(g, _):
        v = vi_ref[0, g]
        x2o_ref[pl.ds(v, 1), :] = v2_s[pl.ds(g, 1), :]
        x3o_ref[pl.ds(v, 1), :] = v3_s[pl.ds(g, 1), :]
        return 0

    lax.fori_loop(0, G, scatter_row, 0)
    h3n_ref[...] = _dot(x3o_ref[...], lin1n_ref[...])
    o = _dot(t, W1_ref[...]) + b1_ref[...]
    o = jnp.maximum(_bn(o, g_ref[...], bb_ref[...]), 0.0)
    out_ref[...] = _dot(o, W2_ref[...]) + b2_ref[...]


def _tc_virt(x2, x3, vi, W1, b1, g, bb, W2, b2, lin1n):
    smem = pl.BlockSpec(memory_space=pltpu.SMEM)
    return pl.pallas_call(
        _tc_virt_body,
        in_specs=[pl.BlockSpec((N, EMB), lambda: (0, 0)),
                  pl.BlockSpec((N, EMB), lambda: (0, 0)),
                  smem,
                  pl.BlockSpec((2 * EMB, 2 * EMB), lambda: (0, 0)),
                  pl.BlockSpec((1, 2 * EMB), lambda: (0, 0)),
                  pl.BlockSpec((1, 2 * EMB), lambda: (0, 0)),
                  pl.BlockSpec((1, 2 * EMB), lambda: (0, 0)),
                  pl.BlockSpec((2 * EMB, 2 * EMB), lambda: (0, 0)),
                  pl.BlockSpec((1, 2 * EMB), lambda: (0, 0)),
                  pl.BlockSpec((EMB, EMB), lambda: (0, 0))],
        scratch_shapes=[pltpu.VMEM((G, EMB), jnp.float32),
                        pltpu.VMEM((G, EMB), jnp.float32)],
        out_shape=[
            jax.ShapeDtypeStruct((N, EMB), jnp.float32),
            jax.ShapeDtypeStruct((N, EMB), jnp.float32),
            jax.ShapeDtypeStruct((N, EMB), jnp.float32),
            jax.ShapeDtypeStruct((G, 2 * EMB), jnp.float32),
        ],
    )(x2, x3, vi, W1, b1, g, bb, W2, b2, lin1n)


def kernel(x, edge_index, edge_attr, positions, batch, atom_emb_2d, atom_emb_3d,
           gin_edge_W, gin_edge_b, gin_W1, gin_b1, gin_W2, gin_b2, bn2d_g,
           bn2d_b, sch_filt_W1, sch_filt_b1, sch_filt_W2, sch_filt_b2,
           sch_lin1_W, sch_lin2_W, sch_lin2_b, sch_out_W, sch_out_b, bn3d_g,
           bn3d_b, int_W1, int_b1, int_bn_g, int_bn_b, int_W2, int_b2):
    f32 = jnp.float32
    row = edge_index[0]
    col = edge_index[1]
    padz = jnp.zeros((E_PAD - E,), jnp.int32)
    padd = jnp.full((E_PAD - E,), DUMMY, jnp.int32)
    row_g = jnp.concatenate([row, padz])
    col_g = jnp.concatenate([col, padz])
    row_s = jnp.concatenate([row, padd])
    col_s = jnp.concatenate([col, padd])
    pos_p = jnp.pad(positions.astype(f32), ((0, 0), (0, 13)))
    attr_p = jnp.pad(edge_attr.astype(f32), ((0, E_PAD - E), (0, 0)))
    emb2_p = jnp.pad(atom_emb_2d.astype(f32), ((0, EMB - 119), (0, 0)))
    emb3_p = jnp.pad(atom_emb_3d.astype(f32), ((0, EMB - 119), (0, 0)))
    fW1_p = jnp.pad(sch_filt_W1.astype(f32), ((0, 0), (0, 14), (0, 0)))

    offs = jnp.pad(jnp.linspace(0.0, CUTOFF, 50).astype(f32), (0, 14))
    offs = offs.reshape(1, 64)
    diff = _sc_posdiff(pos_p, row_g, col_g)
    eemb, wf = _tc_edge(offs, diff, attr_p, gin_edge_W.astype(f32),
                        gin_edge_b.astype(f32), fW1_p, sch_filt_b1.astype(f32),
                        sch_filt_W2.astype(f32), sch_filt_b2.astype(f32))
    x2d, x3d, h3, vi = _tc_init(x.reshape(N, 1), batch.reshape(N, 1),
                                emb2_p, emb3_p, sch_lin1_W[0].astype(f32))
    prev2, prev3 = x2d, x3d

    out = None
    for i in range(NB):
        p = _sc_dualconv(x2d, eemb[i], h3, wf[i], row_g, col_s, col_g, row_s)
        p2 = p[0]
        p3 = p[1]
        x2m = _tc_node2d(x2d, prev2, p2,
                         gin_W1[i].astype(f32), gin_b1[i].reshape(1, 2 * EMB),
                         gin_W2[i].astype(f32), gin_b2[i].reshape(1, EMB),
                         bn2d_g[i].reshape(1, EMB), bn2d_b[i].reshape(1, EMB))
        x3m = _tc_node3d(prev3, p3,
                         sch_lin2_W[i].astype(f32), sch_lin2_b[i].reshape(1, EMB),
                         sch_out_W[i].astype(f32), sch_out_b[i].reshape(1, EMB),
                         bn3d_g[i].reshape(1, EMB), bn3d_b[i].reshape(1, EMB))
        x2d, x3d, h3, out = _tc_virt(
            x2m, x3m, vi, int_W1.astype(f32), int_b1.reshape(1, 2 * EMB),
            int_bn_g.reshape(1, 2 * EMB), int_bn_b.reshape(1, 2 * EMB),
            int_W2.astype(f32), int_b2.reshape(1, 2 * EMB),
            sch_lin1_W[(i + 1) % NB].astype(f32))
    return out
